# baseline (speedup 1.0000x reference)
import jax
import jax.numpy as jnp
from jax import lax
from jax.experimental import pallas as pl
from jax.experimental.pallas import tpu as pltpu
from jax.experimental.pallas import tpu_sc as plsc

VOCAB = 100000
DIM = 64
B = 16384
K = 5

L = 16
NC, NS = 2, 16
NW = NC * NS
S = B // NW
C = 128
NCHUNK = S // C
QD = DIM // L


def _sc_scores(tgt, ctx, posu, posv, negf, scores_out,
               posu_idx, posv_idx, neg_idx, t_rows, cp_rows, neg_rows,
               posd_v, negd_v, sem0, sem1):
    wid = lax.axis_index("s") * NC + lax.axis_index("c")
    sems = [sem0, sem1]

    def stage_and_fire(c, buf):
        base = wid * S + c * C
        pltpu.sync_copy(posu.at[pl.ds(base, C)], posu_idx.at[buf])
        pltpu.sync_copy(posv.at[pl.ds(base, C)], posv_idx.at[buf])
        pltpu.sync_copy(negf.at[pl.ds(base * K, C * K)], neg_idx.at[buf])
        sem = sems[buf]
        cps = [pltpu.async_copy(tgt.at[posu_idx.at[buf]],
                                t_rows.at[buf], sem),
               pltpu.async_copy(ctx.at[posv_idx.at[buf]],
                                cp_rows.at[buf], sem)]
        for j in range(K):
            cps.append(pltpu.async_copy(
                ctx.at[neg_idx.at[buf].at[pl.ds(j * C, C)]],
                neg_rows.at[buf].at[pl.ds(j * C, C)], sem))
        return cps

    def compute(c, buf):
        base = wid * S + c * C
        tb = t_rows.at[buf]
        cb = cp_rows.at[buf]
        nb = neg_rows.at[buf]

        def item_body(i, icarry):
            t = [tb[i, pl.ds(q * L, L)] for q in range(QD)]
            cp = [cb[i, pl.ds(q * L, L)] for q in range(QD)]
            accp = t[0] * cp[0]
            for q in range(1, QD):
                accp = accp + t[q] * cp[q]
            posd_v[pl.ds(i * L, L)] = accp
            cn = [nb[i * K, pl.ds(q * L, L)] for q in range(QD)]
            for k in range(1, K):
                for q in range(QD):
                    cn[q] = cn[q] + nb[i * K + k, pl.ds(q * L, L)]
            accn = t[0] * cn[0]
            for q in range(1, QD):
                accn = accn + t[q] * cn[q]
            negd_v[pl.ds(i * L, L)] = accn
            return icarry

        lax.fori_loop(0, C, item_body, 0)
        pltpu.sync_copy(posd_v, scores_out.at[pl.ds(base * L, C * L)])
        pltpu.sync_copy(negd_v,
                        scores_out.at[pl.ds(B * L + base * L, C * L)])

    handles = {0: stage_and_fire(0, 0)}
    for c in range(NCHUNK):
        buf = c % 2
        if c + 1 < NCHUNK:
            handles[c + 1] = stage_and_fire(c + 1, (c + 1) % 2)
        for h in handles.pop(c):
            h.wait()
        compute(c, buf)


def _tc_loss(scores_ref, out_ref):
    iu = lax.broadcasted_iota(jnp.int32, (128, 128 // L), 0)
    iv = lax.broadcasted_iota(jnp.int32, (128, 128 // L), 1)
    gsum = jnp.where(iu // L == iv, 1.0, 0.0).astype(jnp.float32)
    half = B * L // 128
    yp = jnp.dot(scores_ref[:half, :], gsum,
                 preferred_element_type=jnp.float32)
    yn = jnp.dot(scores_ref[half:, :], gsum,
                 preferred_element_type=jnp.float32)
    s = (jnp.sum(jnp.log(jax.nn.sigmoid(yp)))
         + jnp.sum(jnp.log(jax.nn.sigmoid(-yn))))
    out_ref[0, 0] = -s / jnp.float32(B + B * K)


def kernel(target_table, context_table, pos_u, pos_v, neg_v):
    negf = neg_v.reshape(B * K).astype(jnp.int32)
    pos_u = pos_u.astype(jnp.int32)
    pos_v = pos_v.astype(jnp.int32)

    mesh = plsc.VectorSubcoreMesh(core_axis_name="c", subcore_axis_name="s")
    sc_call = pl.kernel(
        _sc_scores, mesh=mesh,
        compiler_params=pltpu.CompilerParams(use_tc_tiling_on_sc=False),
        out_type=jax.ShapeDtypeStruct((2 * B * L,), jnp.float32),
        scratch_types=[
            pltpu.VMEM((2, C), jnp.int32),
            pltpu.VMEM((2, C), jnp.int32),
            pltpu.VMEM((2, K * C), jnp.int32),
            pltpu.VMEM((2, C, DIM), jnp.float32),
            pltpu.VMEM((2, C, DIM), jnp.float32),
            pltpu.VMEM((2, K * C, DIM), jnp.float32),
            pltpu.VMEM((C * L,), jnp.float32),
            pltpu.VMEM((C * L,), jnp.float32),
            pltpu.SemaphoreType.DMA,
            pltpu.SemaphoreType.DMA,
        ],
    )
    scores = sc_call(target_table, context_table, pos_u, pos_v, negf)

    out = pl.pallas_call(
        _tc_loss,
        out_shape=jax.ShapeDtypeStruct((1, 1), jnp.float32),
        out_specs=pl.BlockSpec(memory_space=pltpu.SMEM),
    )(scores.reshape(2 * B * L // 128, 128))
    return out[0, 0]


# speedup vs baseline: 1.0057x; 1.0057x over previous
import jax
import jax.numpy as jnp
from jax import lax
from jax.experimental import pallas as pl
from jax.experimental.pallas import tpu as pltpu
from jax.experimental.pallas import tpu_sc as plsc

VOCAB = 100000
DIM = 64
B = 16384
K = 5

L = 16
NC, NS = 2, 16
NW = NC * NS
S = B // NW
C = 128
NCHUNK = S // C
QD = DIM // L


def _sc_scores(tgt, ctx, posu, posv, negf, pos_out, neg_out,
               posu_idx, posv_idx, neg_idx, t_rows, cp_rows, neg_rows,
               posd_v, negd_v, sem0, sem1):
    wid = lax.axis_index("s") * NC + lax.axis_index("c")
    sems = [sem0, sem1]

    def stage_and_fire(c, buf):
        base = wid * S + c * C
        pltpu.sync_copy(posu.at[pl.ds(base, C)], posu_idx.at[buf])
        pltpu.sync_copy(posv.at[pl.ds(base, C)], posv_idx.at[buf])
        pltpu.sync_copy(negf.at[pl.ds(base * K, C * K)], neg_idx.at[buf])
        sem = sems[buf]
        cps = [pltpu.async_copy(tgt.at[posu_idx.at[buf]],
                                t_rows.at[buf], sem),
               pltpu.async_copy(ctx.at[posv_idx.at[buf]],
                                cp_rows.at[buf], sem)]
        for j in range(K):
            cps.append(pltpu.async_copy(
                ctx.at[neg_idx.at[buf].at[pl.ds(j * C, C)]],
                neg_rows.at[buf].at[pl.ds(j * C, C)], sem))
        return cps

    def compute(c, buf):
        base = wid * S + c * C
        tb = t_rows.at[buf]
        cb = cp_rows.at[buf]
        nb = neg_rows.at[buf]

        def item_body(i, icarry):
            t = [tb[i, pl.ds(q * L, L)] for q in range(QD)]
            cp = [cb[i, pl.ds(q * L, L)] for q in range(QD)]
            accp = t[0] * cp[0]
            for q in range(1, QD):
                accp = accp + t[q] * cp[q]
            posd_v[pl.ds(i * L, L)] = accp
            cn = [nb[i * K, pl.ds(q * L, L)] for q in range(QD)]
            for k in range(1, K):
                for q in range(QD):
                    cn[q] = cn[q] + nb[i * K + k, pl.ds(q * L, L)]
            accn = t[0] * cn[0]
            for q in range(1, QD):
                accn = accn + t[q] * cn[q]
            negd_v[pl.ds(i * L, L)] = accn
            return icarry

        lax.fori_loop(0, C, item_body, 0)
        pltpu.sync_copy(posd_v, pos_out.at[pl.ds(base * L, C * L)])
        pltpu.sync_copy(negd_v, neg_out.at[pl.ds(base * L, C * L)])

    handles = {0: stage_and_fire(0, 0)}
    for c in range(NCHUNK):
        buf = c % 2
        if c + 1 < NCHUNK:
            handles[c + 1] = stage_and_fire(c + 1, (c + 1) % 2)
        for h in handles.pop(c):
            h.wait()
        compute(c, buf)


def _tc_loss(pos_ref, neg_ref, out_ref):
    iu = lax.broadcasted_iota(jnp.int32, (128, 128 // L), 0)
    iv = lax.broadcasted_iota(jnp.int32, (128, 128 // L), 1)
    gsum = jnp.where(iu // L == iv, 1.0, 0.0).astype(jnp.float32)
    yp = jnp.dot(pos_ref[...], gsum, preferred_element_type=jnp.float32)
    yn = jnp.dot(neg_ref[...], gsum, preferred_element_type=jnp.float32)
    s = (jnp.sum(jnp.log(jax.nn.sigmoid(yp)))
         + jnp.sum(jnp.log(jax.nn.sigmoid(-yn))))
    out_ref[0, 0] = -s / jnp.float32(B + B * K)


def kernel(target_table, context_table, pos_u, pos_v, neg_v):
    negf = neg_v.reshape(B * K).astype(jnp.int32)
    pos_u = pos_u.astype(jnp.int32)
    pos_v = pos_v.astype(jnp.int32)

    mesh = plsc.VectorSubcoreMesh(core_axis_name="c", subcore_axis_name="s")
    sc_call = pl.kernel(
        _sc_scores, mesh=mesh,
        compiler_params=pltpu.CompilerParams(use_tc_tiling_on_sc=False),
        out_type=(jax.ShapeDtypeStruct((B * L,), jnp.float32),
                  jax.ShapeDtypeStruct((B * L,), jnp.float32)),
        scratch_types=[
            pltpu.VMEM((2, C), jnp.int32),
            pltpu.VMEM((2, C), jnp.int32),
            pltpu.VMEM((2, K * C), jnp.int32),
            pltpu.VMEM((2, C, DIM), jnp.float32),
            pltpu.VMEM((2, C, DIM), jnp.float32),
            pltpu.VMEM((2, K * C, DIM), jnp.float32),
            pltpu.VMEM((C * L,), jnp.float32),
            pltpu.VMEM((C * L,), jnp.float32),
            pltpu.SemaphoreType.DMA,
            pltpu.SemaphoreType.DMA,
        ],
    )
    pos_a, neg_a = sc_call(target_table, context_table, pos_u, pos_v, negf)

    out = pl.pallas_call(
        _tc_loss,
        out_shape=jax.ShapeDtypeStruct((1, 1), jnp.float32),
        out_specs=pl.BlockSpec(memory_space=pltpu.SMEM),
    )(pos_a.reshape(B * L // 128, 128), neg_a.reshape(B * L // 128, 128))
    return out[0, 0]
